# baseline (speedup 1.0000x reference)
import functools

import jax
import jax.numpy as jnp
from jax import lax
from jax.experimental import pallas as pl
from jax.experimental.pallas import tpu as pltpu
from jax.experimental.pallas import tpu_sc as plsc

N_NODES = 10000
D = 128
NP = 10240
NC = 2
NS = 16
NW = NC * NS
BATCH = 64
NBUF = 5
ROWS_PER_TILE = NP // NS


def _sc_aggregate(nb, with_counts):
    mesh = plsc.VectorSubcoreMesh(core_axis_name="c", subcore_axis_name="s")

    C = 8
    nch = nb // C
    out_type = [jax.ShapeDtypeStruct((NC, NP, D), jnp.float32)]
    scratch = [
        pltpu.VMEM((3, C, BATCH), jnp.int32),
        pltpu.VMEM((3, C, BATCH), jnp.int32),
        pltpu.VMEM((NBUF, BATCH, D), jnp.float32),
        pltpu.VMEM((8, D), jnp.float32),
        pltpu.VMEM_SHARED((NP, D), jnp.float32),
        pltpu.SemaphoreType.DMA,
        pltpu.SemaphoreType.DMA,
        pltpu.SemaphoreType.DMA,
    ]
    if with_counts:
        out_type.append(jax.ShapeDtypeStruct((NC, NP), jnp.float32))
        scratch += [
            pltpu.VMEM((128,), jnp.float32),
            pltpu.VMEM((BATCH,), jnp.float32),
            pltpu.VMEM_SHARED((NP,), jnp.float32),
            pltpu.SemaphoreType.DMA,
        ]

    @functools.partial(
        pl.kernel,
        out_type=tuple(out_type),
        mesh=mesh,
        scratch_types=scratch,
    )
    def k(x_hbm, src_hbm, dst_hbm, acc_hbm, *rest):
        if with_counts:
            (cnt_hbm, src_v, dst_v, rows_v, zb_v, acc_sh, sem_g, sem_s,
             sem_i, zc_v, ones_v, cnt_sh, sem_c) = rest
        else:
            (src_v, dst_v, rows_v, zb_v, acc_sh, sem_g, sem_s,
             sem_i) = rest
        cid = lax.axis_index("c")
        sid = lax.axis_index("s")
        wid = cid * NS + sid
        r0 = sid * ROWS_PER_TILE

        def issue_idx_load(ch, p):
            rows = pl.ds(pl.multiple_of(wid * nb + ch * C, 8), C)
            pltpu.async_copy(src_hbm.at[rows], src_v.at[p], sem_i)
            pltpu.async_copy(dst_hbm.at[rows], dst_v.at[p], sem_i)

        def wait_idx_load(p):
            rows = pl.ds(wid * nb, C)
            pltpu.make_async_copy(src_hbm.at[rows], src_v.at[p], sem_i).wait()
            pltpu.make_async_copy(dst_hbm.at[rows], dst_v.at[p], sem_i).wait()

        issue_idx_load(0, 0)

        @pl.loop(0, 8)
        def _(r):
            @pl.loop(0, D, step=16)
            def _(c0):
                zb_v[r, pl.ds(c0, 16)] = jnp.zeros((16,), jnp.float32)

        if with_counts:
            @pl.loop(0, 128, step=16)
            def _(i):
                zc_v[pl.ds(i, 16)] = jnp.zeros((16,), jnp.float32)

            @pl.loop(0, BATCH, step=16)
            def _(i):
                ones_v[pl.ds(i, 16)] = jnp.ones((16,), jnp.float32)

        @pl.loop(0, ROWS_PER_TILE, step=8)
        def _(r):
            pltpu.async_copy(zb_v, acc_sh.at[pl.ds(r0 + r, 8)], sem_s)
        if with_counts:
            @pl.loop(0, ROWS_PER_TILE, step=128)
            def _(r):
                pltpu.async_copy(zc_v, cnt_sh.at[pl.ds(r0 + r, 128)], sem_c)

        @pl.loop(0, ROWS_PER_TILE, step=8)
        def _(r):
            pltpu.make_async_copy(zb_v, acc_sh.at[pl.ds(r0, 8)], sem_s).wait()
        if with_counts:
            @pl.loop(0, ROWS_PER_TILE, step=128)
            def _(r):
                pltpu.make_async_copy(zc_v, cnt_sh.at[pl.ds(r0, 128)],
                                      sem_c).wait()

        plsc.subcore_barrier()

        def wait_gather(p):
            pltpu.make_async_copy(x_hbm.at[src_v.at[0, 0]], rows_v.at[p],
                                  sem_g).wait()

        def wait_scatter(p):
            pltpu.make_async_copy(rows_v.at[p], acc_sh.at[dst_v.at[0, 0]],
                                  sem_s).wait()

        def issue_gather(g):
            c = lax.div(g, C)
            q = lax.rem(c, 3)

            @pl.when(lax.rem(g, C) == 0)
            def _():
                wait_idx_load(q)

                @pl.when(c + 1 < nch)
                def _():
                    issue_idx_load(c + 1, lax.rem(c + 1, 3))

            pltpu.async_copy(x_hbm.at[src_v.at[q, lax.rem(g, C)]],
                             rows_v.at[lax.rem(g, NBUF)], sem_g)

        for g0 in range(NBUF - 1):
            issue_gather(jnp.int32(g0))

        @pl.loop(0, nb)
        def _(j):
            r = lax.rem(j, NBUF)
            q = lax.rem(lax.div(j, C), 3)
            jj = lax.rem(j, C)
            wait_gather(r)
            pltpu.async_copy(rows_v.at[r], acc_sh.at[dst_v.at[q, jj]], sem_s,
                             add=True)
            if with_counts:
                pltpu.async_copy(ones_v, cnt_sh.at[dst_v.at[q, jj]], sem_c,
                                 add=True)

            g = j + NBUF - 1

            @pl.when(g < nb)
            def _():
                @pl.when(j >= 1)
                def _():
                    wait_scatter(lax.rem(j - 1, NBUF))

                issue_gather(g)

        @pl.loop(0, min(NBUF, nb))
        def _(k):
            wait_scatter(lax.rem(k, NBUF))

        if with_counts:
            @pl.loop(0, nb)
            def _(j):
                pltpu.make_async_copy(ones_v, cnt_sh.at[dst_v.at[0, 0]],
                                      sem_c).wait()

        plsc.subcore_barrier()

        pltpu.async_copy(acc_sh.at[pl.ds(r0, ROWS_PER_TILE)],
                         acc_hbm.at[cid, pl.ds(r0, ROWS_PER_TILE)], sem_s)
        if with_counts:
            pltpu.async_copy(cnt_sh.at[pl.ds(r0, ROWS_PER_TILE)],
                             cnt_hbm.at[cid, pl.ds(r0, ROWS_PER_TILE)], sem_c)
        pltpu.make_async_copy(acc_sh.at[pl.ds(r0, ROWS_PER_TILE)],
                              acc_hbm.at[cid, pl.ds(r0, ROWS_PER_TILE)],
                              sem_s).wait()
        if with_counts:
            pltpu.make_async_copy(cnt_sh.at[pl.ds(r0, ROWS_PER_TILE)],
                                  cnt_hbm.at[cid, pl.ds(r0, ROWS_PER_TILE)],
                                  sem_c).wait()

    return k


def _tc_combine(acc, cnt3, xp, W_l, b_l2, W_r):
    R = 1024

    def body(acc_ref, cnt_ref, x_ref, wl_ref, bl_ref, wr_ref, o_ref):
        s = acc_ref[0] + acc_ref[1]
        c = cnt_ref[0] + cnt_ref[1]
        mean = s * (1.0 / jnp.maximum(c, 1.0))
        y = lax.dot_general(mean, wl_ref[...],
                            (((1,), (1,)), ((), ())),
                            preferred_element_type=jnp.float32)
        y = y + lax.dot_general(x_ref[...], wr_ref[...],
                                (((1,), (1,)), ((), ())),
                                preferred_element_type=jnp.float32)
        y = y + bl_ref[...]
        o_ref[...] = jnp.where(y >= 0.0, y, 0.01 * y)

    return pl.pallas_call(
        body,
        grid=(NP // R,),
        in_specs=[
            pl.BlockSpec((NC, R, D), lambda i: (0, i, 0)),
            pl.BlockSpec((NC, R, 1), lambda i: (0, i, 0)),
            pl.BlockSpec((R, D), lambda i: (i, 0)),
            pl.BlockSpec((D, D), lambda i: (0, 0)),
            pl.BlockSpec((1, D), lambda i: (0, 0)),
            pl.BlockSpec((D, D), lambda i: (0, 0)),
        ],
        out_specs=pl.BlockSpec((R, D), lambda i: (i, 0)),
        out_shape=jax.ShapeDtypeStruct((NP, D), jnp.float32),
    )(acc, cnt3, xp, W_l, b_l2, W_r)


def kernel(x, edge_index, W_l0, b_l0, W_r0, W_l1, b_l1, W_r1):
    E = edge_index.shape[1]
    nb = -(-E // (NW * BATCH))
    nb = -(-nb // 16) * 16
    e_pad = NW * nb * BATCH

    src = edge_index[0].astype(jnp.int32)
    dst = edge_index[1].astype(jnp.int32)
    src = jnp.pad(src, (0, e_pad - E)).reshape(NW * nb, BATCH)
    dst = jnp.pad(dst, (0, e_pad - E),
                  constant_values=N_NODES).reshape(NW * nb, BATCH)

    xp = jnp.pad(x, ((0, NP - N_NODES), (0, 0)))
    b_l0_2 = b_l0.reshape(1, D)
    b_l1_2 = b_l1.reshape(1, D)

    acc0, cnt = _sc_aggregate(nb, True)(xp, src, dst)
    cnt3 = cnt.reshape(NC, NP, 1)
    h = _tc_combine(acc0, cnt3, xp, W_l0, b_l0_2, W_r0)
    (acc1,) = _sc_aggregate(nb, False)(h, src, dst)
    out = _tc_combine(acc1, cnt3, h, W_l1, b_l1_2, W_r1)
    return out[:N_NODES]


# speedup vs baseline: 1.0024x; 1.0024x over previous
import functools

import jax
import jax.numpy as jnp
from jax import lax
from jax.experimental import pallas as pl
from jax.experimental.pallas import tpu as pltpu
from jax.experimental.pallas import tpu_sc as plsc

N_NODES = 10000
D = 128
NP = 10240
NC = 2
NS = 16
NW = NC * NS
BATCH = 64
NBUF = 4
ROWS_PER_TILE = NP // NS


def _sc_aggregate(nb, with_counts):
    mesh = plsc.VectorSubcoreMesh(core_axis_name="c", subcore_axis_name="s")

    C = 16
    nch = nb // C
    out_type = [jax.ShapeDtypeStruct((NC, NP, D), jnp.float32)]
    scratch = [
        pltpu.VMEM((3, C, BATCH), jnp.int32),
        pltpu.VMEM((3, C, BATCH), jnp.int32),
        pltpu.VMEM((NBUF, BATCH, D), jnp.float32),
        pltpu.VMEM((16, D), jnp.float32),
        pltpu.VMEM_SHARED((NP, D), jnp.float32),
        pltpu.SemaphoreType.DMA,
        pltpu.SemaphoreType.DMA,
        pltpu.SemaphoreType.DMA,
    ]
    if with_counts:
        out_type.append(jax.ShapeDtypeStruct((NC, NP), jnp.float32))
        scratch += [
            pltpu.VMEM((ROWS_PER_TILE,), jnp.float32),
            pltpu.VMEM((BATCH,), jnp.float32),
            pltpu.VMEM_SHARED((NP,), jnp.float32),
            pltpu.SemaphoreType.DMA,
        ]

    @functools.partial(
        pl.kernel,
        out_type=tuple(out_type),
        mesh=mesh,
        scratch_types=scratch,
    )
    def k(x_hbm, src_hbm, dst_hbm, acc_hbm, *rest):
        if with_counts:
            (cnt_hbm, src_v, dst_v, rows_v, zb_v, acc_sh, sem_g, sem_s,
             sem_i, zc_v, ones_v, cnt_sh, sem_c) = rest
        else:
            (src_v, dst_v, rows_v, zb_v, acc_sh, sem_g, sem_s,
             sem_i) = rest
        cid = lax.axis_index("c")
        sid = lax.axis_index("s")
        wid = cid * NS + sid
        r0 = sid * ROWS_PER_TILE

        def issue_idx_load(ch, p):
            rows = pl.ds(pl.multiple_of(wid * nb + ch * C, 16), C)
            pltpu.async_copy(src_hbm.at[rows], src_v.at[p], sem_i)
            pltpu.async_copy(dst_hbm.at[rows], dst_v.at[p], sem_i)

        def wait_idx_load(p):
            rows = pl.ds(wid * nb, C)
            pltpu.make_async_copy(src_hbm.at[rows], src_v.at[p], sem_i).wait()
            pltpu.make_async_copy(dst_hbm.at[rows], dst_v.at[p], sem_i).wait()

        issue_idx_load(0, 0)

        @pl.loop(0, 16)
        def _(r):
            @pl.loop(0, D, step=16)
            def _(c0):
                zb_v[r, pl.ds(c0, 16)] = jnp.zeros((16,), jnp.float32)

        if with_counts:
            @pl.loop(0, ROWS_PER_TILE, step=16)
            def _(i):
                zc_v[pl.ds(i, 16)] = jnp.zeros((16,), jnp.float32)

            @pl.loop(0, BATCH, step=16)
            def _(i):
                ones_v[pl.ds(i, 16)] = jnp.ones((16,), jnp.float32)

        @pl.loop(0, ROWS_PER_TILE, step=16)
        def _(r):
            pltpu.async_copy(zb_v, acc_sh.at[pl.ds(r0 + r, 16)], sem_s)
        if with_counts:
            pltpu.async_copy(zc_v, cnt_sh.at[pl.ds(r0, ROWS_PER_TILE)], sem_c)

        @pl.loop(0, ROWS_PER_TILE, step=16)
        def _(r):
            pltpu.make_async_copy(zb_v, acc_sh.at[pl.ds(r0, 16)], sem_s).wait()
        if with_counts:
            pltpu.make_async_copy(zc_v, cnt_sh.at[pl.ds(r0, ROWS_PER_TILE)],
                                  sem_c).wait()

        plsc.subcore_barrier()

        def wait_gather(p):
            pltpu.make_async_copy(x_hbm.at[src_v.at[0, 0]], rows_v.at[p],
                                  sem_g).wait()

        def wait_scatter(p):
            pltpu.make_async_copy(rows_v.at[p], acc_sh.at[dst_v.at[0, 0]],
                                  sem_s).wait()

        def issue_gather(g):
            c = lax.div(g, C)
            q = lax.rem(c, 3)

            @pl.when(lax.rem(g, C) == 0)
            def _():
                wait_idx_load(q)

                @pl.when(c + 1 < nch)
                def _():
                    issue_idx_load(c + 1, lax.rem(c + 1, 3))

            pltpu.async_copy(x_hbm.at[src_v.at[q, lax.rem(g, C)]],
                             rows_v.at[lax.rem(g, NBUF)], sem_g)

        for g0 in range(NBUF - 1):
            issue_gather(jnp.int32(g0))

        @pl.loop(0, nb)
        def _(j):
            r = lax.rem(j, NBUF)
            q = lax.rem(lax.div(j, C), 3)
            jj = lax.rem(j, C)
            wait_gather(r)
            pltpu.async_copy(rows_v.at[r], acc_sh.at[dst_v.at[q, jj]], sem_s,
                             add=True)
            if with_counts:
                pltpu.async_copy(ones_v, cnt_sh.at[dst_v.at[q, jj]], sem_c,
                                 add=True)

            g = j + NBUF - 1

            @pl.when(g < nb)
            def _():
                @pl.when(j >= 1)
                def _():
                    wait_scatter(lax.rem(j - 1, NBUF))

                issue_gather(g)

        @pl.loop(0, min(NBUF, nb))
        def _(k):
            wait_scatter(lax.rem(k, NBUF))

        if with_counts:
            @pl.loop(0, nb)
            def _(j):
                pltpu.make_async_copy(ones_v, cnt_sh.at[dst_v.at[0, 0]],
                                      sem_c).wait()

        plsc.subcore_barrier()

        pltpu.async_copy(acc_sh.at[pl.ds(r0, ROWS_PER_TILE)],
                         acc_hbm.at[cid, pl.ds(r0, ROWS_PER_TILE)], sem_s)
        if with_counts:
            pltpu.async_copy(cnt_sh.at[pl.ds(r0, ROWS_PER_TILE)],
                             cnt_hbm.at[cid, pl.ds(r0, ROWS_PER_TILE)], sem_c)
        pltpu.make_async_copy(acc_sh.at[pl.ds(r0, ROWS_PER_TILE)],
                              acc_hbm.at[cid, pl.ds(r0, ROWS_PER_TILE)],
                              sem_s).wait()
        if with_counts:
            pltpu.make_async_copy(cnt_sh.at[pl.ds(r0, ROWS_PER_TILE)],
                                  cnt_hbm.at[cid, pl.ds(r0, ROWS_PER_TILE)],
                                  sem_c).wait()

    return k


def _tc_combine(acc, cnt3, xp, W_l, b_l2, W_r):
    R = 1024

    def body(acc_ref, cnt_ref, x_ref, wl_ref, bl_ref, wr_ref, o_ref):
        s = acc_ref[0] + acc_ref[1]
        c = cnt_ref[0] + cnt_ref[1]
        mean = s * (1.0 / jnp.maximum(c, 1.0))
        y = lax.dot_general(mean, wl_ref[...],
                            (((1,), (1,)), ((), ())),
                            preferred_element_type=jnp.float32)
        y = y + lax.dot_general(x_ref[...], wr_ref[...],
                                (((1,), (1,)), ((), ())),
                                preferred_element_type=jnp.float32)
        y = y + bl_ref[...]
        o_ref[...] = jnp.where(y >= 0.0, y, 0.01 * y)

    return pl.pallas_call(
        body,
        grid=(NP // R,),
        in_specs=[
            pl.BlockSpec((NC, R, D), lambda i: (0, i, 0)),
            pl.BlockSpec((NC, R, 1), lambda i: (0, i, 0)),
            pl.BlockSpec((R, D), lambda i: (i, 0)),
            pl.BlockSpec((D, D), lambda i: (0, 0)),
            pl.BlockSpec((1, D), lambda i: (0, 0)),
            pl.BlockSpec((D, D), lambda i: (0, 0)),
        ],
        out_specs=pl.BlockSpec((R, D), lambda i: (i, 0)),
        out_shape=jax.ShapeDtypeStruct((NP, D), jnp.float32),
    )(acc, cnt3, xp, W_l, b_l2, W_r)


def kernel(x, edge_index, W_l0, b_l0, W_r0, W_l1, b_l1, W_r1):
    E = edge_index.shape[1]
    nb = -(-E // (NW * BATCH))
    nb = -(-nb // 16) * 16
    e_pad = NW * nb * BATCH

    src = edge_index[0].astype(jnp.int32)
    dst = edge_index[1].astype(jnp.int32)
    src = jnp.pad(src, (0, e_pad - E)).reshape(NW * nb, BATCH)
    dst = jnp.pad(dst, (0, e_pad - E),
                  constant_values=N_NODES).reshape(NW * nb, BATCH)

    xp = jnp.pad(x, ((0, NP - N_NODES), (0, 0)))
    b_l0_2 = b_l0.reshape(1, D)
    b_l1_2 = b_l1.reshape(1, D)

    acc0, cnt = _sc_aggregate(nb, True)(xp, src, dst)
    cnt3 = cnt.reshape(NC, NP, 1)
    h = _tc_combine(acc0, cnt3, xp, W_l0, b_l0_2, W_r0)
    (acc1,) = _sc_aggregate(nb, False)(h, src, dst)
    out = _tc_combine(acc1, cnt3, h, W_l1, b_l1_2, W_r1)
    return out[:N_NODES]
